# Initial kernel scaffold; baseline (speedup 1.0000x reference)
import functools

import jax
import jax.numpy as jnp
from jax import lax
from jax.experimental import pallas as pl
from jax.experimental.pallas import tpu as pltpu
from jax.experimental.pallas import tpu_sc as plsc

N = 10000
NP = 10112
PADROW = NP - 1
E = 320000
NT = 32
NB = 80
EPT = NB * 128
EPAD = EPT * NT
RB = NP // 16

_MESH = dict(core_axis_name="c", subcore_axis_name="s")


@functools.partial(
    pl.kernel,
    mesh=plsc.VectorSubcoreMesh(**_MESH),
    out_type=[jax.ShapeDtypeStruct((NP,), jnp.float32),
              jax.ShapeDtypeStruct((NP,), jnp.float32)],
    scratch_types=[
        pltpu.VMEM((NB, 128), jnp.int32),
        pltpu.VMEM((128,), jnp.float32),
        pltpu.VMEM_SHARED((NP,), jnp.float32),
    ],
)
def _deg(dst_hbm, ones_hbm, zeros_hbm, out0, out1, idx_d, ones_v, acc):
    c = lax.axis_index("c")
    s = lax.axis_index("s")
    w = c * 16 + s
    pltpu.sync_copy(dst_hbm.at[w], idx_d)
    pltpu.sync_copy(ones_hbm.at[pl.ds(0, 128)], ones_v)
    rs = s * RB

    @pl.when(c == 0)
    def _():
        pltpu.sync_copy(ones_hbm.at[pl.ds(rs, RB)], acc.at[pl.ds(rs, RB)])

    @pl.when(c != 0)
    def _():
        pltpu.sync_copy(zeros_hbm.at[pl.ds(rs, RB)], acc.at[pl.ds(rs, RB)])

    plsc.subcore_barrier()

    def body(j, carry):
        pltpu.sync_copy(ones_v, acc.at[idx_d.at[j]], add=True)
        return carry

    lax.fori_loop(0, NB, body, 0)
    plsc.subcore_barrier()

    @pl.when(c == 0)
    def _():
        pltpu.sync_copy(acc.at[pl.ds(rs, RB)], out0.at[pl.ds(rs, RB)])

    @pl.when(c != 0)
    def _():
        pltpu.sync_copy(acc.at[pl.ds(rs, RB)], out1.at[pl.ds(rs, RB)])


def _make_agg(D):
    @functools.partial(
        pl.kernel,
        mesh=plsc.VectorSubcoreMesh(**_MESH),
        out_type=[jax.ShapeDtypeStruct((NP, D), jnp.float32),
                  jax.ShapeDtypeStruct((NP, D), jnp.float32)],
        scratch_types=[
            pltpu.VMEM((NB, 128), jnp.int32),
            pltpu.VMEM((NB, 128), jnp.int32),
            pltpu.VMEM((128, D), jnp.float32),
            pltpu.VMEM((128, D), jnp.float32),
            pltpu.VMEM_SHARED((NP, D), jnp.float32),
            pltpu.SemaphoreType.DMA,
            pltpu.SemaphoreType.DMA,
        ],
    )
    def agg(src_hbm, dst_hbm, t_hbm, zeros_hbm, out0, out1,
            idx_s, idx_d, buf0, buf1, acc, sem0, sem1):
        c = lax.axis_index("c")
        s = lax.axis_index("s")
        w = c * 16 + s
        pltpu.sync_copy(src_hbm.at[w], idx_s)
        pltpu.sync_copy(dst_hbm.at[w], idx_d)
        rs = s * RB

        @pl.when(c == 0)
        def _():
            pltpu.sync_copy(t_hbm.at[pl.ds(rs, RB)], acc.at[pl.ds(rs, RB)])

        @pl.when(c != 0)
        def _():
            pltpu.sync_copy(zeros_hbm.at[pl.ds(rs, RB)], acc.at[pl.ds(rs, RB)])

        plsc.subcore_barrier()

        pltpu.async_copy(t_hbm.at[idx_s.at[0]], buf0, sem0)
        pltpu.async_copy(t_hbm.at[idx_s.at[1]], buf1, sem1)

        def body(i, carry):
            j0 = 2 * i
            pltpu.make_async_copy(t_hbm.at[idx_s.at[0]], buf0, sem0).wait()
            pltpu.sync_copy(buf0, acc.at[idx_d.at[j0]], add=True)
            pltpu.async_copy(t_hbm.at[idx_s.at[lax.rem(j0 + 2, NB)]], buf0, sem0)
            pltpu.make_async_copy(t_hbm.at[idx_s.at[1]], buf1, sem1).wait()
            pltpu.sync_copy(buf1, acc.at[idx_d.at[j0 + 1]], add=True)
            pltpu.async_copy(t_hbm.at[idx_s.at[lax.rem(j0 + 3, NB)]], buf1, sem1)
            return carry

        lax.fori_loop(0, NB // 2, body, 0)
        pltpu.make_async_copy(t_hbm.at[idx_s.at[0]], buf0, sem0).wait()
        pltpu.make_async_copy(t_hbm.at[idx_s.at[1]], buf1, sem1).wait()
        plsc.subcore_barrier()

        @pl.when(c == 0)
        def _():
            pltpu.sync_copy(acc.at[pl.ds(rs, RB)], out0.at[pl.ds(rs, RB)])

        @pl.when(c != 0)
        def _():
            pltpu.sync_copy(acc.at[pl.ds(rs, RB)], out1.at[pl.ds(rs, RB)])

    return agg


_agg128 = _make_agg(128)
_agg40 = _make_agg(40)


def _first_body(x_ref, w_ref, d0_ref, d1_ref, t_ref, dinv_ref):
    deg = d0_ref[...] + d1_ref[...]
    dinv = lax.rsqrt(deg)
    mm = lax.dot_general(x_ref[...], w_ref[...], (((1,), (0,)), ((), ())),
                         precision=lax.Precision.HIGHEST,
                         preferred_element_type=jnp.float32)
    t_ref[...] = mm * dinv
    dinv_ref[...] = dinv


def _first(xp, W1, d0, d1):
    return pl.pallas_call(
        _first_body,
        grid=(16,),
        in_specs=[
            pl.BlockSpec((RB, 128), lambda i: (i, 0)),
            pl.BlockSpec((128, 128), lambda i: (0, 0)),
            pl.BlockSpec((RB, 1), lambda i: (i, 0)),
            pl.BlockSpec((RB, 1), lambda i: (i, 0)),
        ],
        out_specs=[
            pl.BlockSpec((RB, 128), lambda i: (i, 0)),
            pl.BlockSpec((RB, 1), lambda i: (i, 0)),
        ],
        out_shape=[
            jax.ShapeDtypeStruct((NP, 128), jnp.float32),
            jax.ShapeDtypeStruct((NP, 1), jnp.float32),
        ],
    )(xp, W1, d0, d1)


def _mid_body(a0_ref, a1_ref, dinv_ref, b_ref, w_ref, t_ref):
    dinv = dinv_ref[...]
    act = jnp.maximum((a0_ref[...] + a1_ref[...]) * dinv + b_ref[...], 0.0)
    mm = lax.dot_general(act, w_ref[...], (((1,), (0,)), ((), ())),
                         precision=lax.Precision.HIGHEST,
                         preferred_element_type=jnp.float32)
    t_ref[...] = mm * dinv


def _mid(a0, a1, dinv, b, W, d_out):
    return pl.pallas_call(
        _mid_body,
        grid=(16,),
        in_specs=[
            pl.BlockSpec((RB, 128), lambda i: (i, 0)),
            pl.BlockSpec((RB, 128), lambda i: (i, 0)),
            pl.BlockSpec((RB, 1), lambda i: (i, 0)),
            pl.BlockSpec((128,), lambda i: (0,)),
            pl.BlockSpec((128, d_out), lambda i: (0, 0)),
        ],
        out_specs=pl.BlockSpec((RB, d_out), lambda i: (i, 0)),
        out_shape=jax.ShapeDtypeStruct((NP, d_out), jnp.float32),
    )(a0, a1, dinv, b, W)


def _final_body(a0_ref, a1_ref, dinv_ref, b_ref, o_ref):
    o_ref[...] = (a0_ref[...] + a1_ref[...]) * dinv_ref[...] + b_ref[...]


def _final(a0, a1, dinv, b3):
    return pl.pallas_call(
        _final_body,
        grid=(20,),
        in_specs=[
            pl.BlockSpec((500, 40), lambda i: (i, 0)),
            pl.BlockSpec((500, 40), lambda i: (i, 0)),
            pl.BlockSpec((500, 1), lambda i: (i, 0)),
            pl.BlockSpec((40,), lambda i: (0,)),
        ],
        out_specs=pl.BlockSpec((500, 40), lambda i: (i, 0)),
        out_shape=jax.ShapeDtypeStruct((N, 40), jnp.float32),
    )(a0, a1, dinv, b3)


def kernel(x, edge_index, W1, b1, W2, b2, W3, b3):
    x = x.astype(jnp.float32)
    src = edge_index[0].astype(jnp.int32)
    dst = edge_index[1].astype(jnp.int32)
    pad = jnp.full((EPAD - E,), PADROW, jnp.int32)
    src3 = jnp.concatenate([src, pad]).reshape(NT, NB, 128)
    dst3 = jnp.concatenate([dst, pad]).reshape(NT, NB, 128)

    xp = jnp.pad(x, ((0, NP - N), (0, 0)))
    zeros128 = jnp.zeros((NP, 128), jnp.float32)
    zeros40 = jnp.zeros((NP, 40), jnp.float32)
    ones1 = jnp.ones((NP,), jnp.float32)
    zeros1 = jnp.zeros((NP,), jnp.float32)

    d0, d1 = _deg(dst3, ones1, zeros1)
    t1, dinv = _first(xp, W1, d0.reshape(NP, 1), d1.reshape(NP, 1))
    a0, a1 = _agg128(src3, dst3, t1, zeros128)
    t2 = _mid(a0, a1, dinv, b1, W2, 128)
    a0, a1 = _agg128(src3, dst3, t2, zeros128)
    t3 = _mid(a0, a1, dinv, b2, W3, 40)
    a0, a1 = _agg40(src3, dst3, t3, zeros40)
    return _final(a0, a1, dinv, b3)


# speedup vs baseline: 5.6217x; 5.6217x over previous
import functools

import jax
import jax.numpy as jnp
from jax import lax
from jax.experimental import pallas as pl
from jax.experimental.pallas import tpu as pltpu
from jax.experimental.pallas import tpu_sc as plsc

N = 10000
NP = 10112
PADROW = NP - 1
E = 320000
NT = 32
BS = 128
NB = 80
EPT = NB * BS
EPAD = EPT * NT
RB = NP // 16

_MESH = dict(core_axis_name="c", subcore_axis_name="s")


def _make_agg(D):
    @functools.partial(
        pl.kernel,
        mesh=plsc.VectorSubcoreMesh(**_MESH),
        out_type=[jax.ShapeDtypeStruct((NP, D), jnp.float32),
                  jax.ShapeDtypeStruct((NP, D), jnp.float32)],
        scratch_types=[
            pltpu.VMEM((NB, BS), jnp.int32),
            pltpu.VMEM((NB, BS), jnp.int32),
            pltpu.VMEM((BS, D), jnp.float32),
            pltpu.VMEM_SHARED((NP, D), jnp.float32),
            pltpu.SemaphoreType.DMA,
        ],
    )
    def agg(src_hbm, dst_hbm, t_hbm, zeros_hbm, out0, out1,
            idx_s, idx_d, buf0, acc, sem0):
        c = lax.axis_index("c")
        s = lax.axis_index("s")
        w = c * 16 + s
        pltpu.sync_copy(src_hbm.at[w], idx_s)
        pltpu.sync_copy(dst_hbm.at[w], idx_d)
        rs = s * RB

        @pl.when(c == 0)
        def _():
            pltpu.sync_copy(t_hbm.at[pl.ds(rs, RB)], acc.at[pl.ds(rs, RB)])

        @pl.when(c != 0)
        def _():
            pltpu.sync_copy(zeros_hbm.at[pl.ds(rs, RB)], acc.at[pl.ds(rs, RB)])

        plsc.subcore_barrier()

        def body(j, carry):
            pltpu.async_copy(t_hbm.at[idx_s.at[j]], buf0, sem0).wait()
            pltpu.sync_copy(buf0, acc.at[idx_d.at[j]], add=True)
            return carry

        lax.fori_loop(0, NB, body, 0)
        plsc.subcore_barrier()

        @pl.when(c == 0)
        def _():
            pltpu.sync_copy(acc.at[pl.ds(rs, RB)], out0.at[pl.ds(rs, RB)])

        @pl.when(c != 0)
        def _():
            pltpu.sync_copy(acc.at[pl.ds(rs, RB)], out1.at[pl.ds(rs, RB)])

    return agg


_agg128 = _make_agg(128)


def _first_body(x_ref, w_ref, d0_ref, d1_ref, t_ref, dinv_ref):
    deg = d0_ref[...] + d1_ref[...]
    dinv = lax.rsqrt(deg)
    mm = lax.dot_general(x_ref[...], w_ref[...], (((1,), (0,)), ((), ())),
                         precision=lax.Precision.HIGHEST,
                         preferred_element_type=jnp.float32)
    t_ref[...] = mm * dinv
    dinv_ref[...] = dinv


def _first(xp, W1, d0, d1):
    return pl.pallas_call(
        _first_body,
        grid=(16,),
        in_specs=[
            pl.BlockSpec((RB, 128), lambda i: (i, 0)),
            pl.BlockSpec((128, 128), lambda i: (0, 0)),
            pl.BlockSpec((RB, 1), lambda i: (i, 0)),
            pl.BlockSpec((RB, 1), lambda i: (i, 0)),
        ],
        out_specs=[
            pl.BlockSpec((RB, 128), lambda i: (i, 0)),
            pl.BlockSpec((RB, 1), lambda i: (i, 0)),
        ],
        out_shape=[
            jax.ShapeDtypeStruct((NP, 128), jnp.float32),
            jax.ShapeDtypeStruct((NP, 1), jnp.float32),
        ],
    )(xp, W1, d0, d1)


def _mid_body(a0_ref, a1_ref, dinv_ref, b_ref, w_ref, t_ref):
    dinv = dinv_ref[...]
    act = jnp.maximum((a0_ref[...] + a1_ref[...]) * dinv + b_ref[...], 0.0)
    mm = lax.dot_general(act, w_ref[...], (((1,), (0,)), ((), ())),
                         precision=lax.Precision.HIGHEST,
                         preferred_element_type=jnp.float32)
    t_ref[...] = mm * dinv


def _mid(a0, a1, dinv, b, W, d_out):
    return pl.pallas_call(
        _mid_body,
        grid=(16,),
        in_specs=[
            pl.BlockSpec((RB, 128), lambda i: (i, 0)),
            pl.BlockSpec((RB, 128), lambda i: (i, 0)),
            pl.BlockSpec((RB, 1), lambda i: (i, 0)),
            pl.BlockSpec((128,), lambda i: (0,)),
            pl.BlockSpec((128, d_out), lambda i: (0, 0)),
        ],
        out_specs=pl.BlockSpec((RB, d_out), lambda i: (i, 0)),
        out_shape=jax.ShapeDtypeStruct((NP, d_out), jnp.float32),
    )(a0, a1, dinv, b, W)


def _final_body(a0_ref, a1_ref, dinv_ref, b_ref, o_ref):
    a = a0_ref[...] + a1_ref[...]
    o_ref[...] = a[:, :40] * dinv_ref[...] + b_ref[...]


def _final(a0, a1, dinv, b3):
    return pl.pallas_call(
        _final_body,
        grid=(25,),
        in_specs=[
            pl.BlockSpec((400, 128), lambda i: (i, 0)),
            pl.BlockSpec((400, 128), lambda i: (i, 0)),
            pl.BlockSpec((400, 1), lambda i: (i, 0)),
            pl.BlockSpec((40,), lambda i: (0,)),
        ],
        out_specs=pl.BlockSpec((400, 40), lambda i: (i, 0)),
        out_shape=jax.ShapeDtypeStruct((N, 40), jnp.float32),
    )(a0, a1, dinv, b3)


def kernel(x, edge_index, W1, b1, W2, b2, W3, b3):
    x = x.astype(jnp.float32)
    src = edge_index[0].astype(jnp.int32)
    dst = edge_index[1].astype(jnp.int32)
    pad = jnp.full((EPAD - E,), PADROW, jnp.int32)
    src3 = jnp.concatenate([src, pad]).reshape(NT, NB, BS)
    dst3 = jnp.concatenate([dst, pad]).reshape(NT, NB, BS)

    xp = jnp.pad(x, ((0, NP - N), (0, 0)))
    zeros128 = jnp.zeros((NP, 128), jnp.float32)
    ones128 = jnp.ones((NP, 128), jnp.float32)
    W3p = jnp.pad(W3.astype(jnp.float32), ((0, 0), (0, 128 - 40)))

    g0, g1 = _agg128(src3, dst3, ones128, zeros128)
    t1, dinv = _first(xp, W1, g0[:, :1], g1[:, :1])
    a0, a1 = _agg128(src3, dst3, t1, zeros128)
    t2 = _mid(a0, a1, dinv, b1, W2, 128)
    a0, a1 = _agg128(src3, dst3, t2, zeros128)
    t3 = _mid(a0, a1, dinv, b2, W3p, 128)
    a0, a1 = _agg128(src3, dst3, t3, zeros128)
    return _final(a0, a1, dinv, b3)
